# baseline (speedup 1.0000x reference)
import functools
import jax
import jax.numpy as jnp
from jax import lax
from jax.experimental import pallas as pl
from jax.experimental.pallas import tpu as pltpu
from jax.experimental.pallas import tpu_sc as plsc

B, N, D = 4, 4096, 2048
R = B * N
NC, NS = 2, 16
NW = NC * NS
RPW = R // NW

_mesh = plsc.VectorSubcoreMesh(core_axis_name="c", subcore_axis_name="s")


@functools.partial(
    pl.kernel,
    mesh=_mesh,
    out_type=jax.ShapeDtypeStruct((R * D,), jnp.float32),
    scratch_types=[
        pltpu.SemaphoreType.DMA,
    ],
)
def _flip_rows_sc(x_hbm, out_hbm, sem):
    wid = lax.axis_index("s") * NC + lax.axis_index("c")
    b = wid // (NW // B)
    blk = wid % (NW // B)
    out_base = b * N + blk * RPW
    src_top0 = b * N + (N - 1) - blk * RPW

    K = 16

    def group(gi, _):
        for k in range(K):
            src_off = pl.multiple_of((src_top0 - gi * K - k) * D, D)
            dst_off = pl.multiple_of((out_base + gi * K + k) * D, D)
            pltpu.async_copy(
                x_hbm.at[pl.ds(src_off, D)],
                out_hbm.at[pl.ds(dst_off, D)],
                sem,
            )
        for k in range(K):
            pltpu.make_async_copy(
                x_hbm.at[pl.ds(0, D)], out_hbm.at[pl.ds(0, D)], sem
            ).wait()
        return 0

    lax.fori_loop(0, RPW // K, group, 0)


def kernel(x):
    return _flip_rows_sc(x.reshape(R * D)).reshape(B, N, D)


# speedup vs baseline: 38.7649x; 38.7649x over previous
import functools

import jax
import jax.numpy as jnp
from jax import lax
from jax.experimental import pallas as pl
from jax.experimental.pallas import tpu as pltpu
from jax.experimental.pallas import tpu_sc as plsc

B, N, D = 4, 4096, 2048
R = B * N
NC, NS = 2, 16
NW = NC * NS
RPW = R // NW
C = 16
NCH = RPW // C
NB = 3
NMAIN = (NCH // NB) * NB

_mesh = plsc.VectorSubcoreMesh(core_axis_name="c", subcore_axis_name="s")


@functools.partial(
    pl.kernel,
    mesh=_mesh,
    out_type=jax.ShapeDtypeStruct((R, D), jnp.float32),
    scratch_types=[
        pltpu.VMEM((C,), jnp.int32),
        pltpu.VMEM((C,), jnp.int32),
        pltpu.VMEM((C,), jnp.int32),
        pltpu.VMEM((C, D), jnp.float32),
        pltpu.VMEM((C, D), jnp.float32),
        pltpu.VMEM((C, D), jnp.float32),
        pltpu.SemaphoreType.DMA,
        pltpu.SemaphoreType.DMA,
        pltpu.SemaphoreType.DMA,
        pltpu.SemaphoreType.DMA,
        pltpu.SemaphoreType.DMA,
        pltpu.SemaphoreType.DMA,
    ],
)
def _flip_rows_sc(x_hbm, out_hbm, idx0, idx1, idx2, buf0, buf1, buf2,
                  gs0, gs1, gs2, ws0, ws1, ws2):
    idx = [idx0, idx1, idx2]
    buf = [buf0, buf1, buf2]
    gs = [gs0, gs1, gs2]
    ws = [ws0, ws1, ws2]

    wid = lax.axis_index("s") * NC + lax.axis_index("c")
    b = wid // (NW // B)
    blk = wid % (NW // B)
    out_base = b * N + blk * RPW
    src_top0 = b * N + (N - 1) - blk * RPW

    iota = lax.iota(jnp.int32, 16)

    def start_gather(nb, ci):
        idx[nb][pl.ds(0, 16)] = (src_top0 - ci * C) - iota
        pltpu.async_copy(x_hbm.at[idx[nb]], buf[nb], gs[nb])

    def wait_gather(nb):
        pltpu.make_async_copy(x_hbm.at[idx[nb]], buf[nb], gs[nb]).wait()

    def start_write(nb, ci):
        pltpu.async_copy(buf[nb], out_hbm.at[pl.ds(out_base + ci * C, C)], ws[nb])

    def wait_write(nb):
        pltpu.make_async_copy(buf[nb], out_hbm.at[pl.ds(out_base, C)], ws[nb]).wait()

    for nb in range(NB):
        start_gather(nb, nb)

    def outer(oi, _):
        for nb in range(NB):
            ci = oi * NB + nb

            def traced_step(nb=nb, ci=ci):
                wait_gather(nb)
                start_write(nb, ci)

                @pl.when(ci + NB < NCH)
                def _refill(nb=nb, ci=ci):
                    wait_write(nb)
                    start_gather(nb, ci + NB)

            traced_step()
        return 0

    lax.fori_loop(0, NMAIN // NB, outer, 0)

    for ci in range(NMAIN, NCH):
        nb = ci % NB
        wait_gather(nb)
        start_write(nb, ci)

    for nb in range(NB):
        wait_write(nb)


def kernel(x):
    out = _flip_rows_sc(x.reshape(R, D))
    return out.reshape(B, N, D)
